# Initial kernel scaffold; baseline (speedup 1.0000x reference)
import jax
import jax.numpy as jnp
from jax.experimental import pallas as pl


def kernel(segment_value, segment_lengths, logit, positive_weight, bin_num_examples, bin_num_positives):
    raise NotImplementedError("write your pallas kernel here")


# speedup vs baseline: 2.4497x; 2.4497x over previous
import functools

import jax
import jax.numpy as jnp
from jax import lax
from jax.experimental import pallas as pl
from jax.experimental.pallas import tpu as pltpu
from jax.experimental.pallas import tpu_sc as plsc

_NUM_BINS = 5000
_NUM_LOGITS = 5000
_NUM_SEGMENTS = 42
_NUM_INTERVAL = (_NUM_SEGMENTS + 1) * _NUM_BINS
_BIN_CTR_W = 0.9995
_ONE_MINUS_BIN_CTR_W = 0.0005

_NW = 32
_C = 160
_PAD = _NW * _C
_SL_PAD = _PAD + 8
_LANES = 16


def _sc_body(logit_hbm, seglen_hbm, segval_hbm, bnp_hbm, bne_hbm,
             calib_hbm, bins_hbm,
             logit_v, sl_v, slidx_v, sv_v, bins_v, idx_v, p_v, pos_v, ex_v,
             out_v, sem0, sem1):
    wid = lax.axis_index("s") * 2 + lax.axis_index("c")
    base = wid * _C

    pltpu.sync_copy(logit_hbm.at[pl.ds(base, _C)], logit_v)
    pltpu.sync_copy(seglen_hbm.at[pl.ds(base, _C + 8)], sl_v)

    for j in range(_C // _LANES):
        o = j * _LANES
        x = logit_v[pl.ds(o, _LANES)]
        p = 1.0 / (1.0 + jnp.exp(-x))
        y = p * float(_NUM_BINS)
        t = y.astype(jnp.int32)
        bin0 = jnp.where(y > t.astype(jnp.float32), t, t - 1)
        p_v[pl.ds(o, _LANES)] = p
        bins_v[pl.ds(o, _LANES)] = bin0
        slidx_v[pl.ds(o, _LANES)] = sl_v[pl.ds(o, _LANES)]

    pltpu.async_copy(segval_hbm.at[slidx_v], sv_v, sem0).wait()

    for j in range(_C // _LANES):
        o = j * _LANES
        sl_a = sl_v[pl.ds(o, _LANES)]
        sl_b = sl_v[pl.ds(o + 1, _LANES)]
        dsv = jnp.where(sl_b > sl_a, sv_v[pl.ds(o, _LANES)] + 1, 0)
        dsv = jnp.where(dsv > _NUM_SEGMENTS, 0, dsv)
        bin_id = bins_v[pl.ds(o, _LANES)] + dsv * _NUM_BINS
        bins_v[pl.ds(o, _LANES)] = bin_id
        idx_v[pl.ds(o, _LANES)] = jnp.clip(bin_id, 0, _NUM_INTERVAL - 1)

    cp0 = pltpu.async_copy(bnp_hbm.at[idx_v], pos_v, sem0)
    cp1 = pltpu.async_copy(bne_hbm.at[idx_v], ex_v, sem1)
    cp0.wait()
    cp1.wait()

    for j in range(_C // _LANES):
        o = j * _LANES
        pos = pos_v[pl.ds(o, _LANES)]
        ex = ex_v[pl.ds(o, _LANES)]
        p = p_v[pl.ds(o, _LANES)]
        v = (pos / ex) * _BIN_CTR_W + p * _ONE_MINUS_BIN_CTR_W
        out_v[pl.ds(o, _LANES)] = jnp.where(ex > 0.0, v, p)

    pltpu.sync_copy(out_v, calib_hbm.at[pl.ds(base, _C)])
    pltpu.sync_copy(bins_v, bins_hbm.at[pl.ds(base, _C)])


@jax.jit
def _sc_call(logit_shifted, seglen_pad, segment_value, bnp, bne):
    mesh = plsc.VectorSubcoreMesh(core_axis_name="c", subcore_axis_name="s")
    f = pl.kernel(
        _sc_body,
        out_type=(
            jax.ShapeDtypeStruct((_PAD,), jnp.float32),
            jax.ShapeDtypeStruct((_PAD,), jnp.int32),
        ),
        mesh=mesh,
        scratch_types=[
            pltpu.VMEM((_C,), jnp.float32),
            pltpu.VMEM((_C + 8,), jnp.int32),
            pltpu.VMEM((_C,), jnp.int32),
            pltpu.VMEM((_C,), jnp.int32),
            pltpu.VMEM((_C,), jnp.int32),
            pltpu.VMEM((_C,), jnp.int32),
            pltpu.VMEM((_C,), jnp.float32),
            pltpu.VMEM((_C,), jnp.float32),
            pltpu.VMEM((_C,), jnp.float32),
            pltpu.VMEM((_C,), jnp.float32),
            pltpu.SemaphoreType.DMA,
            pltpu.SemaphoreType.DMA,
        ],
    )
    return f(logit_shifted, seglen_pad, segment_value, bnp, bne)


def kernel(segment_value, segment_lengths, logit, positive_weight,
           bin_num_examples, bin_num_positives):
    logit_shifted = jnp.pad(logit + jnp.log(positive_weight[0]),
                            (0, _PAD - _NUM_LOGITS))
    seglen_pad = jnp.pad(segment_lengths, (0, _SL_PAD - segment_lengths.shape[0]),
                         mode="edge")
    calib, bins = _sc_call(logit_shifted, seglen_pad, segment_value,
                           bin_num_positives, bin_num_examples)
    return calib[:_NUM_LOGITS], bins[:_NUM_LOGITS]


# speedup vs baseline: 2.6722x; 1.0908x over previous
import jax
import jax.numpy as jnp
from jax import lax
from jax.experimental import pallas as pl
from jax.experimental.pallas import tpu as pltpu
from jax.experimental.pallas import tpu_sc as plsc

_NUM_BINS = 5000
_NUM_LOGITS = 5000
_NUM_SEGMENTS = 42
_NUM_INTERVAL = (_NUM_SEGMENTS + 1) * _NUM_BINS
_BIN_CTR_W = 0.9995
_ONE_MINUS_BIN_CTR_W = 0.0005

_NW = 32
_C = 160
_LAST_BASE = _NUM_LOGITS - _C
_LANES = 16


def _sc_body(segval_hbm, seglen_hbm, logit_hbm, pw_hbm, bne_hbm, bnp_hbm,
             calib_hbm, bins_hbm,
             logit_v, sl_v, slidx_v, sv_v, bins_v, idx_v, p_v, pos_v, ex_v,
             out_v, pw_v, s0, s1, s2, s3):
    wid = lax.axis_index("s") * 2 + lax.axis_index("c")
    base = jnp.where(wid == _NW - 1, _LAST_BASE, wid * _C)

    cp_si = pltpu.async_copy(seglen_hbm.at[pl.ds(base, _C)], slidx_v, s0)
    cp_sl = pltpu.async_copy(seglen_hbm.at[pl.ds(base, _C + 1)], sl_v, s1)
    cp_lg = pltpu.async_copy(logit_hbm.at[pl.ds(base, _C)], logit_v, s2)
    cp_pw = pltpu.async_copy(pw_hbm, pw_v.at[pl.ds(0, 1)], s3)
    cp_si.wait()
    cp_sv = pltpu.async_copy(segval_hbm.at[slidx_v], sv_v, s0)
    cp_pw.wait()
    w = pw_v[pl.ds(0, _LANES)][0]
    cp_lg.wait()

    for j in range(_C // _LANES):
        o = j * _LANES
        x = logit_v[pl.ds(o, _LANES)]
        p = w / (w + jnp.exp(-x))
        y = p * float(_NUM_BINS)
        t = y.astype(jnp.int32)
        bin0 = jnp.where(y > t.astype(jnp.float32), t, t - 1)
        p_v[pl.ds(o, _LANES)] = p
        bins_v[pl.ds(o, _LANES)] = bin0

    cp_sl.wait()
    cp_sv.wait()

    for j in range(_C // _LANES):
        o = j * _LANES
        sl_a = sl_v[pl.ds(o, _LANES)]
        sl_b = sl_v[pl.ds(o + 1, _LANES)]
        dsv = jnp.where(sl_b > sl_a, sv_v[pl.ds(o, _LANES)] + 1, 0)
        dsv = jnp.where(dsv > _NUM_SEGMENTS, 0, dsv)
        bin_id = bins_v[pl.ds(o, _LANES)] + dsv * _NUM_BINS
        bins_v[pl.ds(o, _LANES)] = bin_id
        idx_v[pl.ds(o, _LANES)] = jnp.clip(bin_id, 0, _NUM_INTERVAL - 1)

    cp_p = pltpu.async_copy(bnp_hbm.at[idx_v], pos_v, s1)
    cp_e = pltpu.async_copy(bne_hbm.at[idx_v], ex_v, s2)
    cp_p.wait()
    cp_e.wait()

    for j in range(_C // _LANES):
        o = j * _LANES
        pos = pos_v[pl.ds(o, _LANES)]
        ex = ex_v[pl.ds(o, _LANES)]
        p = p_v[pl.ds(o, _LANES)]
        v = (pos / ex) * _BIN_CTR_W + p * _ONE_MINUS_BIN_CTR_W
        out_v[pl.ds(o, _LANES)] = jnp.where(ex > 0.0, v, p)

    co0 = pltpu.async_copy(out_v, calib_hbm.at[pl.ds(base, _C)], s0)
    co1 = pltpu.async_copy(bins_v, bins_hbm.at[pl.ds(base, _C)], s3)
    co0.wait()
    co1.wait()


@jax.jit
def _sc_call(segment_value, segment_lengths, logit, positive_weight, bne, bnp):
    mesh = plsc.VectorSubcoreMesh(core_axis_name="c", subcore_axis_name="s")
    f = pl.kernel(
        _sc_body,
        out_type=(
            jax.ShapeDtypeStruct((_NUM_LOGITS,), jnp.float32),
            jax.ShapeDtypeStruct((_NUM_LOGITS,), jnp.int32),
        ),
        mesh=mesh,
        scratch_types=[
            pltpu.VMEM((_C,), jnp.float32),
            pltpu.VMEM((_C + 1,), jnp.int32),
            pltpu.VMEM((_C,), jnp.int32),
            pltpu.VMEM((_C,), jnp.int32),
            pltpu.VMEM((_C,), jnp.int32),
            pltpu.VMEM((_C,), jnp.int32),
            pltpu.VMEM((_C,), jnp.float32),
            pltpu.VMEM((_C,), jnp.float32),
            pltpu.VMEM((_C,), jnp.float32),
            pltpu.VMEM((_C,), jnp.float32),
            pltpu.VMEM((_LANES,), jnp.float32),
            pltpu.SemaphoreType.DMA,
            pltpu.SemaphoreType.DMA,
            pltpu.SemaphoreType.DMA,
            pltpu.SemaphoreType.DMA,
        ],
    )
    return f(segment_value, segment_lengths, logit, positive_weight, bne, bnp)


def kernel(segment_value, segment_lengths, logit, positive_weight,
           bin_num_examples, bin_num_positives):
    return _sc_call(segment_value, segment_lengths, logit, positive_weight,
                    bin_num_examples, bin_num_positives)


# speedup vs baseline: 2.7019x; 1.0111x over previous
import jax
import jax.numpy as jnp
from jax import lax
from jax.experimental import pallas as pl
from jax.experimental.pallas import tpu as pltpu
from jax.experimental.pallas import tpu_sc as plsc

_NUM_BINS = 5000
_NUM_LOGITS = 5000
_NUM_SEGMENTS = 42
_NUM_INTERVAL = (_NUM_SEGMENTS + 1) * _NUM_BINS
_BIN_CTR_W = 0.9995
_ONE_MINUS_BIN_CTR_W = 0.0005

_NW = 32
_C = 160
_H = _C // 2
_LAST_BASE = _NUM_LOGITS - _C
_LANES = 16


def _sc_body(segval_hbm, seglen_hbm, logit_hbm, pw_hbm, bne_hbm, bnp_hbm,
             calib_hbm, bins_hbm,
             logit_v, sl_v, si_a, si_b, sv_a, sv_b, bins_v, idx_a, idx_b,
             p_v, pos_a, pos_b, ex_a, ex_b, out_v, pw_v,
             s0, s1, s2, s3, s4, s5):
    wid = lax.axis_index("s") * 2 + lax.axis_index("c")
    base = jnp.where(wid == _NW - 1, _LAST_BASE, wid * _C)

    cp_sia = pltpu.async_copy(seglen_hbm.at[pl.ds(base, _H)], si_a, s0)
    cp_sib = pltpu.async_copy(seglen_hbm.at[pl.ds(base + _H, _H)], si_b, s1)
    cp_sl = pltpu.async_copy(seglen_hbm.at[pl.ds(base, _C + 1)], sl_v, s2)
    cp_lg = pltpu.async_copy(logit_hbm.at[pl.ds(base, _C)], logit_v, s3)
    cp_pw = pltpu.async_copy(pw_hbm, pw_v.at[pl.ds(0, 1)], s4)
    cp_sia.wait()
    cp_sva = pltpu.async_copy(segval_hbm.at[si_a], sv_a, s0)
    cp_sib.wait()
    cp_svb = pltpu.async_copy(segval_hbm.at[si_b], sv_b, s1)
    cp_pw.wait()
    w = pw_v[pl.ds(0, _LANES)][0]
    cp_lg.wait()

    for j in range(_C // _LANES):
        o = j * _LANES
        x = logit_v[pl.ds(o, _LANES)]
        p = w / (w + jnp.exp(-x))
        y = p * float(_NUM_BINS)
        t = y.astype(jnp.int32)
        bin0 = jnp.where(y > t.astype(jnp.float32), t, t - 1)
        p_v[pl.ds(o, _LANES)] = p
        bins_v[pl.ds(o, _LANES)] = bin0

    cp_sl.wait()

    def finish_bins(half_off, sv_v, idx_v):
        for j in range(_H // _LANES):
            o = half_off + j * _LANES
            sl_lo = sl_v[pl.ds(o, _LANES)]
            sl_hi = sl_v[pl.ds(o + 1, _LANES)]
            dsv = jnp.where(sl_hi > sl_lo, sv_v[pl.ds(j * _LANES, _LANES)] + 1, 0)
            dsv = jnp.where(dsv > _NUM_SEGMENTS, 0, dsv)
            bin_id = bins_v[pl.ds(o, _LANES)] + dsv * _NUM_BINS
            bins_v[pl.ds(o, _LANES)] = bin_id
            idx_v[pl.ds(j * _LANES, _LANES)] = jnp.clip(bin_id, 0, _NUM_INTERVAL - 1)

    cp_sva.wait()
    finish_bins(0, sv_a, idx_a)
    cp_pa = pltpu.async_copy(bnp_hbm.at[idx_a], pos_a, s3)
    cp_ea = pltpu.async_copy(bne_hbm.at[idx_a], ex_a, s4)

    cp_svb.wait()
    finish_bins(_H, sv_b, idx_b)
    cp_pb = pltpu.async_copy(bnp_hbm.at[idx_b], pos_b, s0)
    cp_eb = pltpu.async_copy(bne_hbm.at[idx_b], ex_b, s1)

    co_bins = pltpu.async_copy(bins_v, bins_hbm.at[pl.ds(base, _C)], s2)

    def blend(half_off, pos_v, ex_v):
        for j in range(_H // _LANES):
            o = half_off + j * _LANES
            pos = pos_v[pl.ds(j * _LANES, _LANES)]
            ex = ex_v[pl.ds(j * _LANES, _LANES)]
            p = p_v[pl.ds(o, _LANES)]
            v = (pos / ex) * _BIN_CTR_W + p * _ONE_MINUS_BIN_CTR_W
            out_v[pl.ds(o, _LANES)] = jnp.where(ex > 0.0, v, p)

    cp_pa.wait()
    cp_ea.wait()
    blend(0, pos_a, ex_a)
    co_cala = pltpu.async_copy(out_v.at[pl.ds(0, _H)],
                               calib_hbm.at[pl.ds(base, _H)], s5)

    cp_pb.wait()
    cp_eb.wait()
    blend(_H, pos_b, ex_b)
    co_calb = pltpu.async_copy(out_v.at[pl.ds(_H, _H)],
                               calib_hbm.at[pl.ds(base + _H, _H)], s3)

    co_bins.wait()
    co_cala.wait()
    co_calb.wait()


@jax.jit
def _sc_call(segment_value, segment_lengths, logit, positive_weight, bne, bnp):
    mesh = plsc.VectorSubcoreMesh(core_axis_name="c", subcore_axis_name="s")
    f = pl.kernel(
        _sc_body,
        out_type=(
            jax.ShapeDtypeStruct((_NUM_LOGITS,), jnp.float32),
            jax.ShapeDtypeStruct((_NUM_LOGITS,), jnp.int32),
        ),
        mesh=mesh,
        scratch_types=[
            pltpu.VMEM((_C,), jnp.float32),
            pltpu.VMEM((_C + 1,), jnp.int32),
            pltpu.VMEM((_H,), jnp.int32),
            pltpu.VMEM((_H,), jnp.int32),
            pltpu.VMEM((_H,), jnp.int32),
            pltpu.VMEM((_H,), jnp.int32),
            pltpu.VMEM((_C,), jnp.int32),
            pltpu.VMEM((_H,), jnp.int32),
            pltpu.VMEM((_H,), jnp.int32),
            pltpu.VMEM((_C,), jnp.float32),
            pltpu.VMEM((_H,), jnp.float32),
            pltpu.VMEM((_H,), jnp.float32),
            pltpu.VMEM((_H,), jnp.float32),
            pltpu.VMEM((_H,), jnp.float32),
            pltpu.VMEM((_C,), jnp.float32),
            pltpu.VMEM((_LANES,), jnp.float32),
            pltpu.SemaphoreType.DMA,
            pltpu.SemaphoreType.DMA,
            pltpu.SemaphoreType.DMA,
            pltpu.SemaphoreType.DMA,
            pltpu.SemaphoreType.DMA,
            pltpu.SemaphoreType.DMA,
        ],
    )
    return f(segment_value, segment_lengths, logit, positive_weight, bne, bnp)


def kernel(segment_value, segment_lengths, logit, positive_weight,
           bin_num_examples, bin_num_positives):
    return _sc_call(segment_value, segment_lengths, logit, positive_weight,
                    bin_num_examples, bin_num_positives)
